# baseline (speedup 1.0000x reference)
import functools

import jax
import jax.numpy as jnp
from jax import lax
from jax.experimental import pallas as pl
from jax.experimental.pallas import tpu as pltpu
from jax.experimental.pallas import tpu_sc as plsc

DIM = 1024
K = 4096
KR = 16
ETA = 0.1
DECAY = 0.01
T = 2048

BT = 256
BK = 512
NT = T // BT
NK = K // BK

_BF = jnp.bfloat16
_F32 = jnp.float32


def _qv_body(x_ref, qvw_ref, wm_ref, wmb_ref, q_ref, v_ref, m_ref):
    x = x_ref[...]
    qv = jax.lax.dot_general(x, qvw_ref[...], (((1,), (1,)), ((), ())),
                             preferred_element_type=_F32)
    q_ref[...] = qv[:, :DIM].astype(_BF)
    v_ref[...] = qv[:, DIM:]
    xw = x.astype(_F32) * wm_ref[...].astype(_F32)
    mlin = jnp.sum(xw, axis=1, keepdims=True)
    m_ref[...] = jax.nn.sigmoid(mlin + wmb_ref[...])


def _logits_body(q_ref, x_ref, s_ref, wnx_ref, wnm_ref, wnb_ref, m_ref,
                 temp_ref, rl_ref, wl_ref):
    s = s_ref[...]
    sf = s.astype(_F32)
    ss = jnp.sum(sf * sf, axis=1)
    scale = 1.0 / (jnp.sqrt(ss) + 1e-6)
    rl = jax.lax.dot_general(q_ref[...], s, (((1,), (1,)), ((), ())),
                             preferred_element_type=_F32)
    rl_ref[...] = rl * (scale[None, :] * temp_ref[...])
    wl = jax.lax.dot_general(x_ref[...], wnx_ref[...], (((1,), (1,)), ((), ())),
                             preferred_element_type=_F32)
    wl_ref[...] = wl + m_ref[...] * wnm_ref[0] + wnb_ref[0]


def _topk_body(l_ref, idx_ref, p_ref):
    l = l_ref[...]
    iota = jax.lax.broadcasted_iota(jnp.int32, (BT, K), 1)
    vals = []
    idxs = []
    for _ in range(KR):
        mx = jnp.max(l, axis=1, keepdims=True)
        cand = jnp.where(l == mx, iota, K)
        am = jnp.min(cand, axis=1, keepdims=True)
        vals.append(mx)
        idxs.append(am)
        l = jnp.where(iota == am, -jnp.inf, l)
    v = jnp.concatenate(vals, axis=1)
    i = jnp.concatenate(idxs, axis=1)
    e = jnp.exp(v - v[:, :1])
    idx_ref[...] = i
    p_ref[...] = e / jnp.sum(e, axis=1, keepdims=True)


def _route_tile(idx, p, kblk):
    kio = jax.lax.broadcasted_iota(jnp.int32, (BT, BK), 1) + kblk * BK
    tile = jnp.zeros((BT, BK), _F32)
    for j in range(KR):
        tile = tile + jnp.where(idx[:, j:j + 1] == kio, p[:, j:j + 1], 0.0)
    return tile.astype(_BF)


def _pre_body(idx_ref, p_ref, s_ref, acc_ref):
    k = pl.program_id(1)
    tile = _route_tile(idx_ref[...], p_ref[...], k)
    contrib = jnp.dot(tile, s_ref[...], preferred_element_type=_F32)

    @pl.when(k == 0)
    def _():
        acc_ref[...] = contrib

    @pl.when(k != 0)
    def _():
        acc_ref[...] = acc_ref[...] + contrib


def _d_body(idx_ref, p_ref, v_ref, pre_ref, d_ref):
    kblk = pl.program_id(0)
    t = pl.program_id(1)
    tile = _route_tile(idx_ref[...], p_ref[...], kblk)
    vo = ((v_ref[...] - pre_ref[...]) * ETA).astype(_BF)
    contrib = jax.lax.dot_general(tile, vo, (((0,), (0,)), ((), ())),
                                  preferred_element_type=_F32)

    @pl.when(t == 0)
    def _():
        d_ref[...] = contrib

    @pl.when(t != 0)
    def _():
        d_ref[...] = d_ref[...] + contrib


def _out_body(idx_ref, p_ref, d_ref, pre_ref, x_ref, o_ref):
    k = pl.program_id(1)
    tile = _route_tile(idx_ref[...], p_ref[...], k)
    contrib = jnp.dot(tile, d_ref[...], preferred_element_type=_F32)

    @pl.when(k == 0)
    def _():
        o_ref[...] = contrib

    @pl.when(k != 0)
    def _():
        o_ref[...] = o_ref[...] + contrib

    @pl.when(k == NK - 1)
    def _():
        o_ref[...] = o_ref[...] + (1.0 - DECAY) * pre_ref[...] + x_ref[...]


_NC = 2
_NS = 16
_NW = _NC * _NS
_TPW = T // _NW
_TB = 4
_NBATCH = _TPW // _TB
_NCH = DIM // 16

_SC_MESH = plsc.VectorSubcoreMesh(core_axis_name="c", subcore_axis_name="s")


def _sc_token_accum(rows_ref, p_ref, orow_ref, pairbase, tt):
    w16 = p_ref[pl.ds(pairbase, KR)]
    ws = [w16.at[jnp.full((16,), j, jnp.int32)].get(mode="promise_in_bounds")
          for j in range(KR)]

    def chunk(c, _):
        sl = pl.ds(c * 16, 16)
        acc = ws[0] * rows_ref[tt * KR, sl]
        for j in range(1, KR):
            acc = acc + ws[j] * rows_ref[tt * KR + j, sl]
        orow_ref[sl] = acc
        return 0

    lax.fori_loop(0, _NCH, chunk, 0)


def _sc_pre_body(s_hbm, idx_hbm, p_hbm, out_hbm, idx_v, p_v, rows_v, orow_v,
                 sem):
    wid = lax.axis_index("s") * _NC + lax.axis_index("c")
    pbase = wid * (_TPW * KR)
    tbase = wid * _TPW
    pltpu.sync_copy(idx_hbm.at[pl.ds(pbase, _TPW * KR)], idx_v)
    pltpu.sync_copy(p_hbm.at[pl.ds(pbase, _TPW * KR)], p_v)

    def batch(b, _):
        pltpu.async_copy(
            s_hbm.at[idx_v.at[pl.ds(b * _TB * KR, _TB * KR)]], rows_v,
            sem).wait()
        for tt in range(_TB):
            tok = b * _TB + tt
            _sc_token_accum(rows_v, p_v, orow_v, tok * KR, tt)
            pltpu.sync_copy(orow_v, out_hbm.at[tbase + tok])
        return 0

    lax.fori_loop(0, _NBATCH, batch, 0)


def _sc_out_body(d_hbm, idx_hbm, p_hbm, pre_hbm, x_hbm, out_hbm, idx_v, p_v,
                 rows_v, orow_v, prow_v, xrow_v, sem):
    wid = lax.axis_index("s") * _NC + lax.axis_index("c")
    pbase = wid * (_TPW * KR)
    tbase = wid * _TPW
    pltpu.sync_copy(idx_hbm.at[pl.ds(pbase, _TPW * KR)], idx_v)
    pltpu.sync_copy(p_hbm.at[pl.ds(pbase, _TPW * KR)], p_v)

    def batch(b, _):
        pltpu.async_copy(
            d_hbm.at[idx_v.at[pl.ds(b * _TB * KR, _TB * KR)]], rows_v,
            sem).wait()
        for tt in range(_TB):
            tok = b * _TB + tt
            pltpu.sync_copy(pre_hbm.at[tbase + tok], prow_v)
            pltpu.sync_copy(x_hbm.at[tbase + tok], xrow_v)
            _sc_token_accum(rows_v, p_v, orow_v, tok * KR, tt)

            def chunk(c, _):
                sl = pl.ds(c * 16, 16)
                orow_v[sl] = (orow_v[sl] + (1.0 - DECAY) * prow_v[sl]
                              + xrow_v[sl])
                return 0

            lax.fori_loop(0, _NCH, chunk, 0)
            pltpu.sync_copy(orow_v, out_hbm.at[tbase + tok])
        return 0

    lax.fori_loop(0, _NBATCH, batch, 0)


def _sc_pre_call(s_f32, ridx, rp):
    return pl.kernel(
        _sc_pre_body,
        mesh=_SC_MESH,
        out_type=jax.ShapeDtypeStruct((T, DIM), _F32),
        scratch_types=[
            pltpu.VMEM((_TPW * KR,), jnp.int32),
            pltpu.VMEM((_TPW * KR,), _F32),
            pltpu.VMEM((_TB * KR, DIM), _F32),
            pltpu.VMEM((DIM,), _F32),
            pltpu.SemaphoreType.DMA,
        ],
    )(s_f32, ridx.reshape(-1), rp.reshape(-1))


def _sc_out_call(d_f32, ridx, rp, pre, x2):
    return pl.kernel(
        _sc_out_body,
        mesh=_SC_MESH,
        out_type=jax.ShapeDtypeStruct((T, DIM), _F32),
        scratch_types=[
            pltpu.VMEM((_TPW * KR,), jnp.int32),
            pltpu.VMEM((_TPW * KR,), _F32),
            pltpu.VMEM((_TB * KR, DIM), _F32),
            pltpu.VMEM((DIM,), _F32),
            pltpu.VMEM((DIM,), _F32),
            pltpu.VMEM((DIM,), _F32),
            pltpu.SemaphoreType.DMA,
        ],
    )(d_f32, ridx.reshape(-1), rp.reshape(-1), pre, x2)


def _topk_call(logits):
    return pl.pallas_call(
        _topk_body,
        grid=(NT,),
        in_specs=[pl.BlockSpec((BT, K), lambda t: (t, 0))],
        out_specs=[pl.BlockSpec((BT, KR), lambda t: (t, 0)),
                   pl.BlockSpec((BT, KR), lambda t: (t, 0))],
        out_shape=[jax.ShapeDtypeStruct((T, KR), jnp.int32),
                   jax.ShapeDtypeStruct((T, KR), _F32)],
    )(logits)


def kernel(x, qkv_w, S_init, read_temperature, Wn_w, Wn_b, Wm_w, Wm_b):
    x2 = x.reshape(T, DIM)
    xb = x2.astype(_BF)
    qvw = jnp.concatenate([qkv_w[:DIM], qkv_w[2 * DIM:]], axis=0).astype(_BF)
    sb = S_init.astype(_BF)
    wnx = Wn_w[:, :DIM].astype(_BF)
    wnm = Wn_w[:, DIM].reshape(NK, 1, BK)
    wnb = Wn_b.reshape(NK, 1, BK)
    wmb = Wm_b.reshape(1, 1)
    wm = Wm_w.astype(_BF)
    temp = read_temperature.reshape(1, 1)

    q, v, m = pl.pallas_call(
        _qv_body,
        grid=(NT,),
        in_specs=[pl.BlockSpec((BT, DIM), lambda t: (t, 0)),
                  pl.BlockSpec((2 * DIM, DIM), lambda t: (0, 0)),
                  pl.BlockSpec((1, DIM), lambda t: (0, 0)),
                  pl.BlockSpec((1, 1), lambda t: (0, 0))],
        out_specs=[pl.BlockSpec((BT, DIM), lambda t: (t, 0)),
                   pl.BlockSpec((BT, DIM), lambda t: (t, 0)),
                   pl.BlockSpec((BT, 1), lambda t: (t, 0))],
        out_shape=[jax.ShapeDtypeStruct((T, DIM), _BF),
                   jax.ShapeDtypeStruct((T, DIM), _F32),
                   jax.ShapeDtypeStruct((T, 1), _F32)],
    )(xb, qvw, wm, wmb)

    rl, wl = pl.pallas_call(
        _logits_body,
        grid=(NT, NK),
        in_specs=[pl.BlockSpec((BT, DIM), lambda t, k: (t, 0)),
                  pl.BlockSpec((BT, DIM), lambda t, k: (t, 0)),
                  pl.BlockSpec((BK, DIM), lambda t, k: (k, 0)),
                  pl.BlockSpec((BK, DIM), lambda t, k: (k, 0)),
                  pl.BlockSpec((1, 1, BK), lambda t, k: (k, 0, 0)),
                  pl.BlockSpec((1, 1, BK), lambda t, k: (k, 0, 0)),
                  pl.BlockSpec((BT, 1), lambda t, k: (t, 0)),
                  pl.BlockSpec((1, 1), lambda t, k: (0, 0))],
        out_specs=[pl.BlockSpec((BT, BK), lambda t, k: (t, k)),
                   pl.BlockSpec((BT, BK), lambda t, k: (t, k))],
        out_shape=[jax.ShapeDtypeStruct((T, K), _F32),
                   jax.ShapeDtypeStruct((T, K), _F32)],
    )(q, xb, sb, wnx, wnm, wnb, m, temp)

    ridx, rp = _topk_call(rl)
    widx, wp = _topk_call(wl)

    pre = _sc_pre_call(S_init, ridx, rp)

    d = pl.pallas_call(
        _d_body,
        grid=(NK, NT),
        in_specs=[pl.BlockSpec((BT, KR), lambda k, t: (t, 0)),
                  pl.BlockSpec((BT, KR), lambda k, t: (t, 0)),
                  pl.BlockSpec((BT, DIM), lambda k, t: (t, 0)),
                  pl.BlockSpec((BT, DIM), lambda k, t: (t, 0))],
        out_specs=pl.BlockSpec((BK, DIM), lambda k, t: (k, 0)),
        out_shape=jax.ShapeDtypeStruct((K, DIM), _F32),
    )(widx, wp, v, pre)

    out = _sc_out_call(d, ridx, rp, pre, x2)

    return out.reshape(x.shape)


# speedup vs baseline: 1.2080x; 1.2080x over previous
import functools

import jax
import jax.numpy as jnp
from jax import lax
from jax.experimental import pallas as pl
from jax.experimental.pallas import tpu as pltpu
from jax.experimental.pallas import tpu_sc as plsc

DIM = 1024
K = 4096
KR = 16
ETA = 0.1
DECAY = 0.01
T = 2048

BT = 256
BK = 512
NT = T // BT
NK = K // BK

_BF = jnp.bfloat16
_F32 = jnp.float32


def _qv_body(x_ref, qvw_ref, wm_ref, wmb_ref, q_ref, v_ref, m_ref):
    x = x_ref[...]
    qv = jax.lax.dot_general(x, qvw_ref[...], (((1,), (1,)), ((), ())),
                             preferred_element_type=_F32)
    q_ref[...] = qv[:, :DIM].astype(_BF)
    v_ref[...] = qv[:, DIM:]
    xw = x.astype(_F32) * wm_ref[...].astype(_F32)
    mlin = jnp.sum(xw, axis=1, keepdims=True)
    m_ref[...] = jax.nn.sigmoid(mlin + wmb_ref[...])


def _logits_body(q_ref, x_ref, s_ref, wnx_ref, wnm_ref, wnb_ref, m_ref,
                 temp_ref, rl_ref, wl_ref):
    s = s_ref[...]
    sf = s.astype(_F32)
    ss = jnp.sum(sf * sf, axis=1)
    scale = 1.0 / (jnp.sqrt(ss) + 1e-6)
    rl = jax.lax.dot_general(q_ref[...], s, (((1,), (1,)), ((), ())),
                             preferred_element_type=_F32)
    rl_ref[...] = rl * (scale[None, :] * temp_ref[...])
    wl = jax.lax.dot_general(x_ref[...], wnx_ref[...], (((1,), (1,)), ((), ())),
                             preferred_element_type=_F32)
    wl_ref[...] = wl + m_ref[...] * wnm_ref[0] + wnb_ref[0]


def _topk_body(l_ref, idx_ref, p_ref):
    l = l_ref[...]
    iota = jax.lax.broadcasted_iota(jnp.int32, (BT, K), 1)
    vals = []
    idxs = []
    for _ in range(KR):
        mx = jnp.max(l, axis=1, keepdims=True)
        cand = jnp.where(l == mx, iota, K)
        am = jnp.min(cand, axis=1, keepdims=True)
        vals.append(mx)
        idxs.append(am)
        l = jnp.where(iota == am, -jnp.inf, l)
    v = jnp.concatenate(vals, axis=1)
    i = jnp.concatenate(idxs, axis=1)
    e = jnp.exp(v - v[:, :1])
    idx_ref[...] = i
    p_ref[...] = e / jnp.sum(e, axis=1, keepdims=True)


def _route_tile(idx, p, kblk):
    kio = jax.lax.broadcasted_iota(jnp.int32, (BT, BK), 1) + kblk * BK
    tile = jnp.zeros((BT, BK), _F32)
    for j in range(KR):
        tile = tile + jnp.where(idx[:, j:j + 1] == kio, p[:, j:j + 1], 0.0)
    return tile.astype(_BF)


def _pre_body(idx_ref, p_ref, s_ref, acc_ref):
    k = pl.program_id(1)
    tile = _route_tile(idx_ref[...], p_ref[...], k)
    contrib = jnp.dot(tile, s_ref[...], preferred_element_type=_F32)

    @pl.when(k == 0)
    def _():
        acc_ref[...] = contrib

    @pl.when(k != 0)
    def _():
        acc_ref[...] = acc_ref[...] + contrib


def _d_body(idx_ref, p_ref, v_ref, pre_ref, d_ref):
    kblk = pl.program_id(0)
    t = pl.program_id(1)
    tile = _route_tile(idx_ref[...], p_ref[...], kblk)
    vo = ((v_ref[...] - pre_ref[...]) * ETA).astype(_BF)
    contrib = jax.lax.dot_general(tile, vo, (((0,), (0,)), ((), ())),
                                  preferred_element_type=_F32)

    @pl.when(t == 0)
    def _():
        d_ref[...] = contrib

    @pl.when(t != 0)
    def _():
        d_ref[...] = d_ref[...] + contrib


def _out_body(idx_ref, p_ref, d_ref, pre_ref, x_ref, o_ref):
    k = pl.program_id(1)
    tile = _route_tile(idx_ref[...], p_ref[...], k)
    contrib = jnp.dot(tile, d_ref[...], preferred_element_type=_F32)

    @pl.when(k == 0)
    def _():
        o_ref[...] = contrib

    @pl.when(k != 0)
    def _():
        o_ref[...] = o_ref[...] + contrib

    @pl.when(k == NK - 1)
    def _():
        o_ref[...] = o_ref[...] + (1.0 - DECAY) * pre_ref[...] + x_ref[...]


_NC = 2
_NS = 16
_NW = _NC * _NS
_TPW = T // _NW
_TB = 2
_NBATCH = _TPW // _TB
_NCH = DIM // 16

_SC_MESH = plsc.VectorSubcoreMesh(core_axis_name="c", subcore_axis_name="s")


def _sc_token_accum(rows_ref, p_ref, orow_ref, pairbase, tt):
    w16 = p_ref[pl.ds(pairbase, KR)]
    ws = [w16.at[jnp.full((16,), j, jnp.int32)].get(mode="promise_in_bounds")
          for j in range(KR)]

    def chunk(c, _):
        sl = pl.ds(c * 16, 16)
        acc = ws[0] * rows_ref[tt * KR, sl]
        for j in range(1, KR):
            acc = acc + ws[j] * rows_ref[tt * KR + j, sl]
        orow_ref[sl] = acc
        return 0

    lax.fori_loop(0, _NCH, chunk, 0)


_GROWS = _TB * KR
_NPAIR = _NBATCH // 2


def _sc_gather_body(tab_hbm, idx_hbm, p_hbm, out_hbm, idx_v, p_v, rows_a,
                    rows_b, orow_v, sem_a, sem_b):
    wid = lax.axis_index("s") * _NC + lax.axis_index("c")
    pbase = wid * (_TPW * KR)
    tbase = wid * _TPW
    pltpu.sync_copy(idx_hbm.at[pl.ds(pbase, _TPW * KR)], idx_v)
    pltpu.sync_copy(p_hbm.at[pl.ds(pbase, _TPW * KR)], p_v)

    def start(b, buf, sem):
        pltpu.async_copy(tab_hbm.at[idx_v.at[pl.ds(b * _GROWS, _GROWS)]],
                         buf, sem)

    def drain(buf, sem):
        pltpu.make_async_copy(tab_hbm.at[pl.ds(0, _GROWS)], buf, sem).wait()

    def compute(b, buf):
        for tt in range(_TB):
            tok = b * _TB + tt
            _sc_token_accum(buf, p_v, orow_v, tok * KR, tt)
            pltpu.sync_copy(orow_v, out_hbm.at[tbase + tok])

    start(0, rows_a, sem_a)

    def pair(g, _):
        b0 = 2 * g
        b1 = b0 + 1
        start(b1, rows_b, sem_b)
        drain(rows_a, sem_a)
        compute(b0, rows_a)

        @pl.when(b1 + 1 < _NBATCH)
        def _():
            start(b1 + 1, rows_a, sem_a)

        drain(rows_b, sem_b)
        compute(b1, rows_b)
        return 0

    lax.fori_loop(0, _NPAIR, pair, 0)


def _sc_gather_call(table_f32, idx, p):
    return pl.kernel(
        _sc_gather_body,
        mesh=_SC_MESH,
        out_type=jax.ShapeDtypeStruct((T, DIM), _F32),
        scratch_types=[
            pltpu.VMEM((_TPW * KR,), jnp.int32),
            pltpu.VMEM((_TPW * KR,), _F32),
            pltpu.VMEM((_GROWS, DIM), _F32),
            pltpu.VMEM((_GROWS, DIM), _F32),
            pltpu.VMEM((DIM,), _F32),
            pltpu.SemaphoreType.DMA,
            pltpu.SemaphoreType.DMA,
        ],
    )(table_f32, idx.reshape(-1), p.reshape(-1))


def _fin_body(rd_ref, pre_ref, x_ref, o_ref):
    o_ref[...] = rd_ref[...] + (1.0 - DECAY) * pre_ref[...] + x_ref[...]


def _topk_call(logits):
    return pl.pallas_call(
        _topk_body,
        grid=(NT,),
        in_specs=[pl.BlockSpec((BT, K), lambda t: (t, 0))],
        out_specs=[pl.BlockSpec((BT, KR), lambda t: (t, 0)),
                   pl.BlockSpec((BT, KR), lambda t: (t, 0))],
        out_shape=[jax.ShapeDtypeStruct((T, KR), jnp.int32),
                   jax.ShapeDtypeStruct((T, KR), _F32)],
    )(logits)


def kernel(x, qkv_w, S_init, read_temperature, Wn_w, Wn_b, Wm_w, Wm_b):
    x2 = x.reshape(T, DIM)
    xb = x2.astype(_BF)
    qvw = jnp.concatenate([qkv_w[:DIM], qkv_w[2 * DIM:]], axis=0).astype(_BF)
    sb = S_init.astype(_BF)
    wnx = Wn_w[:, :DIM].astype(_BF)
    wnm = Wn_w[:, DIM].reshape(NK, 1, BK)
    wnb = Wn_b.reshape(NK, 1, BK)
    wmb = Wm_b.reshape(1, 1)
    wm = Wm_w.astype(_BF)
    temp = read_temperature.reshape(1, 1)

    q, v, m = pl.pallas_call(
        _qv_body,
        grid=(NT,),
        in_specs=[pl.BlockSpec((BT, DIM), lambda t: (t, 0)),
                  pl.BlockSpec((2 * DIM, DIM), lambda t: (0, 0)),
                  pl.BlockSpec((1, DIM), lambda t: (0, 0)),
                  pl.BlockSpec((1, 1), lambda t: (0, 0))],
        out_specs=[pl.BlockSpec((BT, DIM), lambda t: (t, 0)),
                   pl.BlockSpec((BT, DIM), lambda t: (t, 0)),
                   pl.BlockSpec((BT, 1), lambda t: (t, 0))],
        out_shape=[jax.ShapeDtypeStruct((T, DIM), _BF),
                   jax.ShapeDtypeStruct((T, DIM), _F32),
                   jax.ShapeDtypeStruct((T, 1), _F32)],
    )(xb, qvw, wm, wmb)

    rl, wl = pl.pallas_call(
        _logits_body,
        grid=(NT, NK),
        in_specs=[pl.BlockSpec((BT, DIM), lambda t, k: (t, 0)),
                  pl.BlockSpec((BT, DIM), lambda t, k: (t, 0)),
                  pl.BlockSpec((BK, DIM), lambda t, k: (k, 0)),
                  pl.BlockSpec((BK, DIM), lambda t, k: (k, 0)),
                  pl.BlockSpec((1, 1, BK), lambda t, k: (k, 0, 0)),
                  pl.BlockSpec((1, 1, BK), lambda t, k: (k, 0, 0)),
                  pl.BlockSpec((BT, 1), lambda t, k: (t, 0)),
                  pl.BlockSpec((1, 1), lambda t, k: (0, 0))],
        out_specs=[pl.BlockSpec((BT, BK), lambda t, k: (t, k)),
                   pl.BlockSpec((BT, BK), lambda t, k: (t, k))],
        out_shape=[jax.ShapeDtypeStruct((T, K), _F32),
                   jax.ShapeDtypeStruct((T, K), _F32)],
    )(q, xb, sb, wnx, wnm, wnb, m, temp)

    ridx, rp = _topk_call(rl)
    widx, wp = _topk_call(wl)

    pre = _sc_gather_call(S_init, ridx, rp)

    d = pl.pallas_call(
        _d_body,
        grid=(NK, NT),
        in_specs=[pl.BlockSpec((BT, KR), lambda k, t: (t, 0)),
                  pl.BlockSpec((BT, KR), lambda k, t: (t, 0)),
                  pl.BlockSpec((BT, DIM), lambda k, t: (t, 0)),
                  pl.BlockSpec((BT, DIM), lambda k, t: (t, 0))],
        out_specs=pl.BlockSpec((BK, DIM), lambda k, t: (k, 0)),
        out_shape=jax.ShapeDtypeStruct((K, DIM), _F32),
    )(widx, wp, v, pre)

    rd = _sc_gather_call(d, ridx, rp)

    out = pl.pallas_call(
        _fin_body,
        grid=(NT,),
        in_specs=[pl.BlockSpec((BT, DIM), lambda t: (t, 0)),
                  pl.BlockSpec((BT, DIM), lambda t: (t, 0)),
                  pl.BlockSpec((BT, DIM), lambda t: (t, 0))],
        out_specs=pl.BlockSpec((BT, DIM), lambda t: (t, 0)),
        out_shape=jax.ShapeDtypeStruct((T, DIM), _F32),
    )(rd, pre, x2)

    return out.reshape(x.shape)


# speedup vs baseline: 1.3045x; 1.0799x over previous
import functools

import jax
import jax.numpy as jnp
from jax import lax
from jax.experimental import pallas as pl
from jax.experimental.pallas import tpu as pltpu
from jax.experimental.pallas import tpu_sc as plsc

DIM = 1024
K = 4096
KR = 16
ETA = 0.1
DECAY = 0.01
T = 2048

BT = 256
BK = 512
NT = T // BT
NK = K // BK

_BF = jnp.bfloat16
_F32 = jnp.float32


def _proj_body(x_ref, qvw_ref, wm_ref, wmb_ref, s_ref, wnx_ref, wnm_ref,
               wnb_ref, temp_ref, rl_ref, wl_ref, v_ref, q_s, m_s):
    k = pl.program_id(1)
    x = x_ref[...]

    @pl.when(k == 0)
    def _():
        qv = jax.lax.dot_general(x, qvw_ref[...], (((1,), (1,)), ((), ())),
                                 preferred_element_type=_F32)
        q_s[...] = qv[:, :DIM].astype(_BF)
        v_ref[...] = qv[:, DIM:]
        xw = x.astype(_F32) * wm_ref[...].astype(_F32)
        mlin = jnp.sum(xw, axis=1, keepdims=True)
        m_s[...] = jax.nn.sigmoid(mlin + wmb_ref[...])

    s = s_ref[...]
    sf = s.astype(_F32)
    ss = jnp.sum(sf * sf, axis=1)
    scale = 1.0 / (jnp.sqrt(ss) + 1e-6)
    rl = jax.lax.dot_general(q_s[...], s, (((1,), (1,)), ((), ())),
                             preferred_element_type=_F32)
    rl_ref[...] = rl * (scale[None, :] * temp_ref[...])
    wl = jax.lax.dot_general(x, wnx_ref[...], (((1,), (1,)), ((), ())),
                             preferred_element_type=_F32)
    wl_ref[...] = wl + m_s[...] * wnm_ref[0] + wnb_ref[0]


def _topk_body(l_ref, idx_ref, p_ref):
    l = l_ref[...]
    iota = jax.lax.broadcasted_iota(jnp.int32, (BT, K), 1).astype(_F32)
    vals = []
    idxs = []
    for _ in range(KR):
        mx = jnp.max(l, axis=1, keepdims=True)
        cand = jnp.where(l == mx, iota, 1e9)
        am = jnp.min(cand, axis=1, keepdims=True)
        vals.append(mx)
        idxs.append(am)
        l = jnp.where(cand == am, -jnp.inf, l)
    v = jnp.concatenate(vals, axis=1)
    i = jnp.concatenate(idxs, axis=1)
    e = jnp.exp(v - v[:, :1])
    idx_ref[...] = i.astype(jnp.int32)
    p_ref[...] = e / jnp.sum(e, axis=1, keepdims=True)


def _route_tile(idx, p, kblk):
    kio = jax.lax.broadcasted_iota(jnp.int32, (BT, BK), 1) + kblk * BK
    tile = jnp.zeros((BT, BK), _F32)
    for j in range(KR):
        tile = tile + jnp.where(idx[:, j:j + 1] == kio, p[:, j:j + 1], 0.0)
    return tile.astype(_BF)


def _pre_body(idx_ref, p_ref, s_ref, acc_ref):
    k = pl.program_id(1)
    tile = _route_tile(idx_ref[...], p_ref[...], k)
    contrib = jnp.dot(tile, s_ref[...], preferred_element_type=_F32)

    @pl.when(k == 0)
    def _():
        acc_ref[...] = contrib

    @pl.when(k != 0)
    def _():
        acc_ref[...] = acc_ref[...] + contrib


def _d_body(idx_ref, p_ref, v_ref, pre_ref, d_ref):
    kblk = pl.program_id(0)
    t = pl.program_id(1)
    tile = _route_tile(idx_ref[...], p_ref[...], kblk)
    vo = ((v_ref[...] - pre_ref[...]) * ETA).astype(_BF)
    contrib = jax.lax.dot_general(tile, vo, (((0,), (0,)), ((), ())),
                                  preferred_element_type=_F32)

    @pl.when(t == 0)
    def _():
        d_ref[...] = contrib

    @pl.when(t != 0)
    def _():
        d_ref[...] = d_ref[...] + contrib


def _out_body(idx_ref, p_ref, d_ref, pre_ref, x_ref, o_ref):
    k = pl.program_id(1)
    tile = _route_tile(idx_ref[...], p_ref[...], k)
    contrib = jnp.dot(tile, d_ref[...], preferred_element_type=_F32)

    @pl.when(k == 0)
    def _():
        o_ref[...] = contrib

    @pl.when(k != 0)
    def _():
        o_ref[...] = o_ref[...] + contrib

    @pl.when(k == NK - 1)
    def _():
        o_ref[...] = o_ref[...] + (1.0 - DECAY) * pre_ref[...] + x_ref[...]


_NC = 2
_NS = 16
_NW = _NC * _NS
_TPW = T // _NW
_TB = 2
_NBATCH = _TPW // _TB
_NCH = DIM // 16

_SC_MESH = plsc.VectorSubcoreMesh(core_axis_name="c", subcore_axis_name="s",
                                  num_cores=_NC, num_subcores=_NS)


def _sc_token_accum(rows_ref, p_ref, orow_ref, pairbase, tt):
    w16 = p_ref[pl.ds(pairbase, KR)]
    ws = [w16.at[jnp.full((16,), j, jnp.int32)].get(mode="promise_in_bounds")
          for j in range(KR)]

    def chunk(c, _):
        sl = pl.ds(c * 16, 16)
        acc = ws[0] * rows_ref[tt * KR, sl]
        for j in range(1, KR):
            acc = acc + ws[j] * rows_ref[tt * KR + j, sl]
        orow_ref[sl] = acc
        return 0

    lax.fori_loop(0, _NCH, chunk, 0)


_GROWS = _TB * KR
_NPAIR = _NBATCH // 2


def _sc_gather_body(tab_hbm, idx_hbm, p_hbm, out_hbm, idx_v, p_v, rows_a,
                    rows_b, orow_v, sem_a, sem_b):
    wid = lax.axis_index("s") * _NC + lax.axis_index("c")
    pbase = wid * (_TPW * KR)
    tbase = wid * _TPW
    pltpu.sync_copy(idx_hbm.at[pl.ds(pbase, _TPW * KR)], idx_v)
    pltpu.sync_copy(p_hbm.at[pl.ds(pbase, _TPW * KR)], p_v)

    def start(b, buf, sem):
        pltpu.async_copy(tab_hbm.at[idx_v.at[pl.ds(b * _GROWS, _GROWS)]],
                         buf, sem)

    def drain(buf, sem):
        pltpu.make_async_copy(tab_hbm.at[pl.ds(0, _GROWS)], buf, sem).wait()

    def compute(b, buf):
        for tt in range(_TB):
            tok = b * _TB + tt
            _sc_token_accum(buf, p_v, orow_v, tok * KR, tt)
            pltpu.sync_copy(orow_v, out_hbm.at[tbase + tok])

    start(0, rows_a, sem_a)

    def pair(g, _):
        b0 = 2 * g
        b1 = b0 + 1
        start(b1, rows_b, sem_b)
        drain(rows_a, sem_a)
        compute(b0, rows_a)

        @pl.when(b1 + 1 < _NBATCH)
        def _():
            start(b1 + 1, rows_a, sem_a)

        drain(rows_b, sem_b)
        compute(b1, rows_b)
        return 0

    lax.fori_loop(0, _NPAIR, pair, 0)


def _sc_gather_call(table_f32, idx, p):
    return pl.kernel(
        _sc_gather_body,
        mesh=_SC_MESH,
        out_type=jax.ShapeDtypeStruct((T, DIM), _F32),
        scratch_types=[
            pltpu.VMEM((_TPW * KR,), jnp.int32),
            pltpu.VMEM((_TPW * KR,), _F32),
            pltpu.VMEM((_GROWS, DIM), _F32),
            pltpu.VMEM((_GROWS, DIM), _F32),
            pltpu.VMEM((DIM,), _F32),
            pltpu.SemaphoreType.DMA,
            pltpu.SemaphoreType.DMA,
        ],
    )(table_f32, idx.reshape(-1), p.reshape(-1))


def _fin_body(rd_ref, pre_ref, x_ref, o_ref):
    o_ref[...] = rd_ref[...] + (1.0 - DECAY) * pre_ref[...] + x_ref[...]


def _topk_call(logits):
    return pl.pallas_call(
        _topk_body,
        grid=(NT,),
        in_specs=[pl.BlockSpec((BT, K), lambda t: (t, 0))],
        out_specs=[pl.BlockSpec((BT, KR), lambda t: (t, 0)),
                   pl.BlockSpec((BT, KR), lambda t: (t, 0))],
        out_shape=[jax.ShapeDtypeStruct((T, KR), jnp.int32),
                   jax.ShapeDtypeStruct((T, KR), _F32)],
    )(logits)


def kernel(x, qkv_w, S_init, read_temperature, Wn_w, Wn_b, Wm_w, Wm_b):
    x2 = x.reshape(T, DIM)
    xb = x2.astype(_BF)
    qvw = jnp.concatenate([qkv_w[:DIM], qkv_w[2 * DIM:]], axis=0).astype(_BF)
    sb = S_init.astype(_BF)
    wnx = Wn_w[:, :DIM].astype(_BF)
    wnm = Wn_w[:, DIM].reshape(NK, 1, BK)
    wnb = Wn_b.reshape(NK, 1, BK)
    wmb = Wm_b.reshape(1, 1)
    wm = Wm_w.astype(_BF)
    temp = read_temperature.reshape(1, 1)

    rl, wl, v = pl.pallas_call(
        _proj_body,
        grid=(NT, NK),
        in_specs=[pl.BlockSpec((BT, DIM), lambda t, k: (t, 0)),
                  pl.BlockSpec((2 * DIM, DIM), lambda t, k: (0, 0)),
                  pl.BlockSpec((1, DIM), lambda t, k: (0, 0)),
                  pl.BlockSpec((1, 1), lambda t, k: (0, 0)),
                  pl.BlockSpec((BK, DIM), lambda t, k: (k, 0)),
                  pl.BlockSpec((BK, DIM), lambda t, k: (k, 0)),
                  pl.BlockSpec((1, 1, BK), lambda t, k: (k, 0, 0)),
                  pl.BlockSpec((1, 1, BK), lambda t, k: (k, 0, 0)),
                  pl.BlockSpec((1, 1), lambda t, k: (0, 0))],
        out_specs=[pl.BlockSpec((BT, BK), lambda t, k: (t, k)),
                   pl.BlockSpec((BT, BK), lambda t, k: (t, k)),
                   pl.BlockSpec((BT, DIM), lambda t, k: (t, 0))],
        out_shape=[jax.ShapeDtypeStruct((T, K), _F32),
                   jax.ShapeDtypeStruct((T, K), _F32),
                   jax.ShapeDtypeStruct((T, DIM), _F32)],
        scratch_shapes=[pltpu.VMEM((BT, DIM), _BF),
                        pltpu.VMEM((BT, 1), _F32)],
    )(xb, qvw, wm, wmb, sb, wnx, wnm, wnb, temp)

    ridx, rp = _topk_call(rl)
    pre = _sc_gather_call(S_init, ridx, rp)
    widx, wp = _topk_call(wl)

    d = pl.pallas_call(
        _d_body,
        grid=(NK, NT),
        in_specs=[pl.BlockSpec((BT, KR), lambda k, t: (t, 0)),
                  pl.BlockSpec((BT, KR), lambda k, t: (t, 0)),
                  pl.BlockSpec((BT, DIM), lambda k, t: (t, 0)),
                  pl.BlockSpec((BT, DIM), lambda k, t: (t, 0))],
        out_specs=pl.BlockSpec((BK, DIM), lambda k, t: (k, 0)),
        out_shape=jax.ShapeDtypeStruct((K, DIM), _F32),
    )(widx, wp, v, pre)

    rd = _sc_gather_call(d, ridx, rp)

    out = pl.pallas_call(
        _fin_body,
        grid=(NT,),
        in_specs=[pl.BlockSpec((BT, DIM), lambda t: (t, 0)),
                  pl.BlockSpec((BT, DIM), lambda t: (t, 0)),
                  pl.BlockSpec((BT, DIM), lambda t: (t, 0))],
        out_specs=pl.BlockSpec((BT, DIM), lambda t: (t, 0)),
        out_shape=jax.ShapeDtypeStruct((T, DIM), _F32),
    )(rd, pre, x2)

    return out.reshape(x.shape)


# speedup vs baseline: 1.3701x; 1.0503x over previous
import functools

import jax
import jax.numpy as jnp
from jax import lax
from jax.experimental import pallas as pl
from jax.experimental.pallas import tpu as pltpu
from jax.experimental.pallas import tpu_sc as plsc

DIM = 1024
K = 4096
KR = 16
ETA = 0.1
DECAY = 0.01
T = 2048

BT = 256
BK = 512
NT = T // BT
NK = K // BK

_BF = jnp.bfloat16
_F32 = jnp.float32


def _proj_body(x_ref, qvw_ref, wm_ref, wmb_ref, s_ref, wnx_ref, wnm_ref,
               wnb_ref, temp_ref, rl_ref, wl_ref, v_ref, q_s, m_s):
    k = pl.program_id(1)
    x = x_ref[...]

    @pl.when(k == 0)
    def _():
        qv = jax.lax.dot_general(x, qvw_ref[...], (((1,), (1,)), ((), ())),
                                 preferred_element_type=_F32)
        q_s[...] = qv[:, :DIM].astype(_BF)
        v_ref[...] = qv[:, DIM:]
        xw = x.astype(_F32) * wm_ref[...].astype(_F32)
        mlin = jnp.sum(xw, axis=1, keepdims=True)
        m_s[...] = jax.nn.sigmoid(mlin + wmb_ref[...])

    s = s_ref[...]
    sf = s.astype(_F32)
    ss = jnp.sum(sf * sf, axis=1)
    scale = 1.0 / (jnp.sqrt(ss) + 1e-6)
    rl = jax.lax.dot_general(q_s[...], s, (((1,), (1,)), ((), ())),
                             preferred_element_type=_F32)
    rl_ref[...] = rl * (scale[None, :] * temp_ref[...])
    wl = jax.lax.dot_general(x, wnx_ref[...], (((1,), (1,)), ((), ())),
                             preferred_element_type=_F32)
    wl_ref[...] = wl + m_s[...] * wnm_ref[0] + wnb_ref[0]


def _topk_body(l_ref, idx_ref, p_ref):
    s = jax.lax.bitcast_convert_type(l_ref[...], jnp.int32)
    mono = s ^ ((s >> 31) & 0x7FFFFFFF)
    kt = mono & (-4096)
    iota = jax.lax.broadcasted_iota(jnp.int32, (BT, K), 1)
    keys = kt | (4095 - iota)
    kvals = []
    for _ in range(KR):
        mx = jnp.max(keys, axis=1, keepdims=True)
        kvals.append(mx)
        keys = jnp.where(keys == mx, jnp.int32(-2 ** 31), keys)
    km = jnp.concatenate(kvals, axis=1)
    idx = 4095 - (km & 4095)
    vt = km & (-4096)
    v = jax.lax.bitcast_convert_type(vt ^ ((vt >> 31) & 0x7FFFFFFF), _F32)
    e = jnp.exp(v - v[:, :1])
    idx_ref[...] = idx
    p_ref[...] = e / jnp.sum(e, axis=1, keepdims=True)


def _route_tile(idx, p, kblk):
    kio = jax.lax.broadcasted_iota(jnp.int32, (BT, BK), 1) + kblk * BK
    tile = jnp.zeros((BT, BK), _F32)
    for j in range(KR):
        tile = tile + jnp.where(idx[:, j:j + 1] == kio, p[:, j:j + 1], 0.0)
    return tile.astype(_BF)


def _pre_body(idx_ref, p_ref, s_ref, acc_ref):
    k = pl.program_id(1)
    tile = _route_tile(idx_ref[...], p_ref[...], k)
    contrib = jnp.dot(tile, s_ref[...], preferred_element_type=_F32)

    @pl.when(k == 0)
    def _():
        acc_ref[...] = contrib

    @pl.when(k != 0)
    def _():
        acc_ref[...] = acc_ref[...] + contrib


def _d_body(idx_ref, p_ref, v_ref, pre_ref, d_ref):
    kblk = pl.program_id(0)
    t = pl.program_id(1)
    tile = _route_tile(idx_ref[...], p_ref[...], kblk)
    vo = ((v_ref[...] - pre_ref[...]) * ETA).astype(_BF)
    contrib = jax.lax.dot_general(tile, vo, (((0,), (0,)), ((), ())),
                                  preferred_element_type=_F32)

    @pl.when(t == 0)
    def _():
        d_ref[...] = contrib

    @pl.when(t != 0)
    def _():
        d_ref[...] = d_ref[...] + contrib


def _out_body(idx_ref, p_ref, d_ref, pre_ref, x_ref, o_ref):
    k = pl.program_id(1)
    tile = _route_tile(idx_ref[...], p_ref[...], k)
    contrib = jnp.dot(tile, d_ref[...], preferred_element_type=_F32)

    @pl.when(k == 0)
    def _():
        o_ref[...] = contrib

    @pl.when(k != 0)
    def _():
        o_ref[...] = o_ref[...] + contrib

    @pl.when(k == NK - 1)
    def _():
        o_ref[...] = o_ref[...] + (1.0 - DECAY) * pre_ref[...] + x_ref[...]


_NC = 2
_NS = 16
_NW = _NC * _NS
_TPW = T // _NW
_TB = 2
_NBATCH = _TPW // _TB
_NCH = DIM // 16

@functools.cache
def _sc_mesh():
    return plsc.VectorSubcoreMesh(core_axis_name="c", subcore_axis_name="s",
                                  num_cores=_NC, num_subcores=_NS)


def _sc_token_accum(rows_ref, p_ref, orow_ref, pairbase, tt):
    w16 = p_ref[pl.ds(pairbase, KR)]
    ws = [w16.at[jnp.full((16,), j, jnp.int32)].get(mode="promise_in_bounds")
          for j in range(KR)]

    def chunk(c, _):
        sl = pl.ds(c * 16, 16)
        acc = ws[0] * rows_ref[tt * KR, sl]
        for j in range(1, KR):
            acc = acc + ws[j] * rows_ref[tt * KR + j, sl]
        orow_ref[sl] = acc
        return 0

    lax.fori_loop(0, _NCH, chunk, 0)


_GROWS = _TB * KR
_NPAIR = _NBATCH // 2


def _sc_gather_body(tab_hbm, idx_hbm, p_hbm, out_hbm, idx_v, p_v, rows_a,
                    rows_b, orow_v, sem_a, sem_b):
    wid = lax.axis_index("s") * _NC + lax.axis_index("c")
    pbase = wid * (_TPW * KR)
    tbase = wid * _TPW
    pltpu.sync_copy(idx_hbm.at[pl.ds(pbase, _TPW * KR)], idx_v)
    pltpu.sync_copy(p_hbm.at[pl.ds(pbase, _TPW * KR)], p_v)

    def start(b, buf, sem):
        pltpu.async_copy(tab_hbm.at[idx_v.at[pl.ds(b * _GROWS, _GROWS)]],
                         buf, sem)

    def drain(buf, sem):
        pltpu.make_async_copy(tab_hbm.at[pl.ds(0, _GROWS)], buf, sem).wait()

    def compute(b, buf):
        for tt in range(_TB):
            tok = b * _TB + tt
            _sc_token_accum(buf, p_v, orow_v, tok * KR, tt)
            pltpu.sync_copy(orow_v, out_hbm.at[tbase + tok])

    start(0, rows_a, sem_a)

    def pair(g, _):
        b0 = 2 * g
        b1 = b0 + 1
        start(b1, rows_b, sem_b)
        drain(rows_a, sem_a)
        compute(b0, rows_a)

        @pl.when(b1 + 1 < _NBATCH)
        def _():
            start(b1 + 1, rows_a, sem_a)

        drain(rows_b, sem_b)
        compute(b1, rows_b)
        return 0

    lax.fori_loop(0, _NPAIR, pair, 0)


def _sc_gather_call(table_f32, idx, p):
    return pl.kernel(
        _sc_gather_body,
        mesh=_sc_mesh(),
        out_type=jax.ShapeDtypeStruct((T, DIM), _F32),
        scratch_types=[
            pltpu.VMEM((_TPW * KR,), jnp.int32),
            pltpu.VMEM((_TPW * KR,), _F32),
            pltpu.VMEM((_GROWS, DIM), _F32),
            pltpu.VMEM((_GROWS, DIM), _F32),
            pltpu.VMEM((DIM,), _F32),
            pltpu.SemaphoreType.DMA,
            pltpu.SemaphoreType.DMA,
        ],
    )(table_f32, idx.reshape(-1), p.reshape(-1))


def _fin_body(rd_ref, pre_ref, x_ref, o_ref):
    o_ref[...] = rd_ref[...] + (1.0 - DECAY) * pre_ref[...] + x_ref[...]


def _topk_call(logits):
    return pl.pallas_call(
        _topk_body,
        grid=(NT,),
        in_specs=[pl.BlockSpec((BT, K), lambda t: (t, 0))],
        out_specs=[pl.BlockSpec((BT, KR), lambda t: (t, 0)),
                   pl.BlockSpec((BT, KR), lambda t: (t, 0))],
        out_shape=[jax.ShapeDtypeStruct((T, KR), jnp.int32),
                   jax.ShapeDtypeStruct((T, KR), _F32)],
    )(logits)


def kernel(x, qkv_w, S_init, read_temperature, Wn_w, Wn_b, Wm_w, Wm_b):
    x2 = x.reshape(T, DIM)
    xb = x2.astype(_BF)
    qvw = jnp.concatenate([qkv_w[:DIM], qkv_w[2 * DIM:]], axis=0).astype(_BF)
    sb = S_init.astype(_BF)
    wnx = Wn_w[:, :DIM].astype(_BF)
    wnm = Wn_w[:, DIM].reshape(NK, 1, BK)
    wnb = Wn_b.reshape(NK, 1, BK)
    wmb = Wm_b.reshape(1, 1)
    wm = Wm_w.astype(_BF)
    temp = read_temperature.reshape(1, 1)

    rl, wl, v = pl.pallas_call(
        _proj_body,
        grid=(NT, NK),
        in_specs=[pl.BlockSpec((BT, DIM), lambda t, k: (t, 0)),
                  pl.BlockSpec((2 * DIM, DIM), lambda t, k: (0, 0)),
                  pl.BlockSpec((1, DIM), lambda t, k: (0, 0)),
                  pl.BlockSpec((1, 1), lambda t, k: (0, 0)),
                  pl.BlockSpec((BK, DIM), lambda t, k: (k, 0)),
                  pl.BlockSpec((BK, DIM), lambda t, k: (k, 0)),
                  pl.BlockSpec((1, 1, BK), lambda t, k: (k, 0, 0)),
                  pl.BlockSpec((1, 1, BK), lambda t, k: (k, 0, 0)),
                  pl.BlockSpec((1, 1), lambda t, k: (0, 0))],
        out_specs=[pl.BlockSpec((BT, BK), lambda t, k: (t, k)),
                   pl.BlockSpec((BT, BK), lambda t, k: (t, k)),
                   pl.BlockSpec((BT, DIM), lambda t, k: (t, 0))],
        out_shape=[jax.ShapeDtypeStruct((T, K), _F32),
                   jax.ShapeDtypeStruct((T, K), _F32),
                   jax.ShapeDtypeStruct((T, DIM), _F32)],
        scratch_shapes=[pltpu.VMEM((BT, DIM), _BF),
                        pltpu.VMEM((BT, 1), _F32)],
    )(xb, qvw, wm, wmb, sb, wnx, wnm, wnb, temp)

    ridx, rp = _topk_call(rl)
    pre = _sc_gather_call(S_init, ridx, rp)
    widx, wp = _topk_call(wl)

    d = pl.pallas_call(
        _d_body,
        grid=(NK, NT),
        in_specs=[pl.BlockSpec((BT, KR), lambda k, t: (t, 0)),
                  pl.BlockSpec((BT, KR), lambda k, t: (t, 0)),
                  pl.BlockSpec((BT, DIM), lambda k, t: (t, 0)),
                  pl.BlockSpec((BT, DIM), lambda k, t: (t, 0))],
        out_specs=pl.BlockSpec((BK, DIM), lambda k, t: (k, 0)),
        out_shape=jax.ShapeDtypeStruct((K, DIM), _F32),
    )(widx, wp, v, pre)

    rd = _sc_gather_call(d, ridx, rp)

    out = pl.pallas_call(
        _fin_body,
        grid=(NT,),
        in_specs=[pl.BlockSpec((BT, DIM), lambda t: (t, 0)),
                  pl.BlockSpec((BT, DIM), lambda t: (t, 0)),
                  pl.BlockSpec((BT, DIM), lambda t: (t, 0))],
        out_specs=pl.BlockSpec((BT, DIM), lambda t: (t, 0)),
        out_shape=jax.ShapeDtypeStruct((T, DIM), _F32),
    )(rd, pre, x2)

    return out.reshape(x.shape)
